# baseline (speedup 1.0000x reference)
import jax
import jax.numpy as jnp
from jax.experimental import pallas as pl

_BN = 16384
_C = 128


def _tmk_block(xt_ref, ptsb_ref, c_ref, out_ref):
    D = xt_ref.shape[0] - 1
    M = c_ref.shape[0]
    bn = out_ref.shape[0]
    c = c_ref[...]
    for j in range(bn // _C):
        sx = xt_ref[D : D + 1, j * _C : (j + 1) * _C]
        acc0 = -(ptsb_ref[D * M : (D + 1) * M, :] + sx)
        acc1 = None
        for d in range(D):
            xr = xt_ref[d : d + 1, j * _C : (j + 1) * _C]
            pb = ptsb_ref[d * M : (d + 1) * M, :]
            t = jnp.minimum(pb, xr)
            if d % 2 == 0:
                acc0 = acc0 + t
            else:
                acc1 = t if acc1 is None else acc1 + t
        kt = jnp.exp(acc0 + acc1)
        out_ref[j * _C : (j + 1) * _C, :] = jax.lax.dot_general(
            kt, c, (((0,), (0,)), ((), ())), preferred_element_type=jnp.float32
        )


def kernel(input, pts_set, chol_inv):
    N, D = input.shape
    M = pts_set.shape[0]
    bn = min(_BN, N)
    while N % bn:
        bn //= 2
    xt = jnp.concatenate(
        [input * 2.0, input.sum(1, keepdims=True)], axis=1
    ).T
    pa = jnp.concatenate(
        [pts_set * 2.0, pts_set.sum(1, keepdims=True)], axis=1
    )
    pts_b = jnp.broadcast_to(pa.T[:, :, None], (D + 1, M, _C)).reshape(
        (D + 1) * M, _C
    )
    return pl.pallas_call(
        _tmk_block,
        grid=(N // bn,),
        in_specs=[
            pl.BlockSpec((D + 1, bn), lambda i: (0, i)),
            pl.BlockSpec(((D + 1) * M, _C), lambda i: (0, 0)),
            pl.BlockSpec((M, M), lambda i: (0, 0)),
        ],
        out_specs=pl.BlockSpec((bn, M), lambda i: (i, 0)),
        out_shape=jax.ShapeDtypeStruct((N, M), jnp.float32),
    )(xt, pts_b, chol_inv)


# speedup vs baseline: 1.0073x; 1.0073x over previous
import jax
import jax.numpy as jnp
from jax.experimental import pallas as pl

_BN = 16384
_C = 128


def _tmk_block(xt_ref, ptsb_ref, c_ref, out_ref):
    D = xt_ref.shape[0] - 1
    M = c_ref.shape[0]
    bn = out_ref.shape[0]
    c = c_ref[...]
    for j in range(bn // _C):
        sx = xt_ref[D : D + 1, j * _C : (j + 1) * _C]
        acc = -(ptsb_ref[D * M : (D + 1) * M, :] + sx)
        for d in range(D):
            xr = xt_ref[d : d + 1, j * _C : (j + 1) * _C]
            pb = ptsb_ref[d * M : (d + 1) * M, :]
            acc = acc + jnp.minimum(pb, xr)
        kt = jnp.exp(acc)
        out_ref[j * _C : (j + 1) * _C, :] = jax.lax.dot_general(
            kt, c, (((0,), (0,)), ((), ())), preferred_element_type=jnp.float32
        )


def kernel(input, pts_set, chol_inv):
    N, D = input.shape
    M = pts_set.shape[0]
    bn = min(_BN, N)
    while N % bn:
        bn //= 2
    xt = jnp.concatenate(
        [input * 2.0, input.sum(1, keepdims=True)], axis=1
    ).T
    pa = jnp.concatenate(
        [pts_set * 2.0, pts_set.sum(1, keepdims=True)], axis=1
    )
    pts_b = jnp.broadcast_to(pa.T[:, :, None], (D + 1, M, _C)).reshape(
        (D + 1) * M, _C
    )
    return pl.pallas_call(
        _tmk_block,
        grid=(N // bn,),
        in_specs=[
            pl.BlockSpec((D + 1, bn), lambda i: (0, i)),
            pl.BlockSpec(((D + 1) * M, _C), lambda i: (0, 0)),
            pl.BlockSpec((M, M), lambda i: (0, 0)),
        ],
        out_specs=pl.BlockSpec((bn, M), lambda i: (i, 0)),
        out_shape=jax.ShapeDtypeStruct((N, M), jnp.float32),
    )(xt, pts_b, chol_inv)
